# baseline (speedup 1.0000x reference)
import jax
import jax.numpy as jnp
from jax import lax
from jax.experimental import pallas as pl
from jax.experimental.pallas import tpu as pltpu
from jax.experimental.pallas import tpu_sc as plsc

N = 10000
H = 256
E2 = 80000
EP = 81920
NP = 10240
LANES = 16
_SEG_CHUNK = 128
_SC_CHUNK = 64

_GATHER_DNUMS = lax.GatherDimensionNumbers(
    offset_dims=(), collapsed_slice_dims=(0,), start_index_map=(0,))


def _lane_shuffle(v, idx):
    return lax.gather(v, idx[:, None], _GATHER_DNUMS, (1,),
                      mode=lax.GatherScatterMode.PROMISE_IN_BOUNDS)


def _dense_pre_body(emb_ref, f_ref, l1_ref, l3_ref, fcw_ref, linb_ref, fcb2_ref,
                    a_ref, c_ref, fcf_ref):
    a_ref[...] = (jnp.dot(emb_ref[...], l1_ref[...],
                          preferred_element_type=jnp.float32)
                  + linb_ref[...]).astype(jnp.bfloat16)
    c_ref[...] = jnp.dot(f_ref[...], l3_ref[...],
                         preferred_element_type=jnp.float32).astype(jnp.bfloat16)
    fcf_ref[...] = jnp.dot(f_ref[...], fcw_ref[...],
                           preferred_element_type=jnp.float32) + fcb2_ref[...]


def _dense_post_body(srlo_ref, srhi_ref, dega_ref, degb_ref, emb_ref, fcf_ref,
                     l1a_ref, l1b_ref, l2_ref, linb_ref, p_ref):
    deg = dega_ref[:, 0:1] + degb_ref[:, 0:1]
    recv = deg > 0.0
    denom = jnp.maximum(deg, 1.0)
    rlo = jnp.where(recv, srlo_ref[...] / denom, emb_ref[:, :128])
    rhi = jnp.where(recv, srhi_ref[...] / denom, emb_ref[:, 128:])
    m1 = jnp.where(recv, jnp.maximum(fcf_ref[...], 0.0), 0.0)
    p_ref[...] = (jnp.dot(rlo, l1a_ref[...], preferred_element_type=jnp.float32)
                  + jnp.dot(rhi, l1b_ref[...], preferred_element_type=jnp.float32)
                  + jnp.dot(m1, l2_ref[...], preferred_element_type=jnp.float32)
                  + linb_ref[...]).astype(jnp.bfloat16)


def _segment_body(emb2_hbm, s3_hbm, d3_hbm,
                  sr_hbm, deg_hbm,
                  acc, sidx_all, didx_all, rows_a, rows_b, sem_a, sem_b):
    c = lax.axis_index("c")
    s = lax.axis_index("s")
    n_chunks = EP // 16 // _SEG_CHUNK

    def _init(r, carry):
        for j in range(8):
            rows_a[r, pl.ds(16 * j, 16)] = jnp.zeros((16,), jnp.float32)
        return carry

    lax.fori_loop(0, _SEG_CHUNK, _init, 0)

    for i in range(5):
        pltpu.sync_copy(rows_a, acc.at[pl.ds(s * 640 + i * 128, 128)])

    pltpu.sync_copy(s3_hbm.at[s], sidx_all)
    pltpu.sync_copy(d3_hbm.at[s], didx_all)
    row_off = c * N

    def _off(j, carry):
        for i in range(_SEG_CHUNK // 16):
            sl = pl.ds(16 * i, 16)
            sidx_all[j, sl] = sidx_all[j, sl] + row_off
        return carry

    lax.fori_loop(0, n_chunks, _off, 0)
    plsc.subcore_barrier()

    def _start(buf, sem, ch):
        pltpu.async_copy(emb2_hbm.at[sidx_all.at[ch]], buf, sem)

    def _finish(buf, sem, ch):
        pltpu.make_async_copy(emb2_hbm.at[sidx_all.at[ch]], buf, sem).wait()
        pltpu.sync_copy(buf, acc.at[didx_all.at[ch]], add=True)

    _start(rows_a, sem_a, 0)

    def _pair(k, carry):
        _start(rows_b, sem_b, 2 * k + 1)
        _finish(rows_a, sem_a, 2 * k)

        @pl.when(k < n_chunks // 2 - 1)
        def _():
            _start(rows_a, sem_a, 2 * k + 2)

        _finish(rows_b, sem_b, 2 * k + 1)
        return carry

    lax.fori_loop(0, n_chunks // 2, _pair, 0)
    plsc.subcore_barrier()

    rb = s * (NP // 16)
    pltpu.sync_copy(acc.at[pl.ds(rb, NP // 16)],
                    sr_hbm.at[c, pl.ds(rb, NP // 16)])

    def _fill(r, carry):
        for j in range(8):
            rows_b[r, pl.ds(16 * j, 16)] = jnp.zeros((16,), jnp.float32)
            rows_a[r, pl.ds(16 * j, 16)] = jnp.ones((16,), jnp.float32)
        return carry

    lax.fori_loop(0, _SEG_CHUNK, _fill, 0)
    for i in range(5):
        pltpu.sync_copy(rows_b, acc.at[pl.ds(s * 640 + i * 128, 128)])
    plsc.subcore_barrier()

    def _count(i, carry):
        ch = c * (n_chunks // 2) + i
        pltpu.sync_copy(rows_a, acc.at[didx_all.at[ch]], add=True)
        return carry

    lax.fori_loop(0, n_chunks // 2, _count, 0)
    plsc.subcore_barrier()
    pltpu.sync_copy(acc.at[pl.ds(rb, NP // 16)],
                    deg_hbm.at[c, pl.ds(rb, NP // 16)])


def _make_score_body():
    chunk = 64
    n_chunks = (EP // 32) // chunk

    def body(*refs):
        (table_hbm, c_hbm, s_hbm, d_hbm, usw_hbm, usb_hbm,
         out_hbm,
         usv, usbv, sidx_all, didx_all,
         a_a, c_a, a_b, c_b, outbuf, sem_a, sem_b) = refs

        cc = lax.axis_index("c")
        ss = lax.axis_index("s")
        w = ss * 2 + cc
        base = w * (EP // 32)

        pltpu.sync_copy(usw_hbm, usv)
        pltpu.sync_copy(usb_hbm, usbv)
        pltpu.sync_copy(s_hbm.at[pl.ds(base, EP // 32)], sidx_all)
        pltpu.sync_copy(d_hbm.at[pl.ds(base, EP // 32)], didx_all)
        us = [usv[pl.ds(16 * j, 16)] for j in range(LANES)]
        lane_iota = lax.broadcasted_iota(jnp.int32, (16,), 0)
        usb_all = _lane_shuffle(usbv[...], lane_iota * 0)

        def _start(abuf, cbuf, sem, ch):
            sl = pl.ds(ch * chunk, chunk)
            pltpu.async_copy(table_hbm.at[sidx_all.at[sl]], abuf, sem)
            pltpu.async_copy(c_hbm.at[didx_all.at[sl]], cbuf, sem)

        def _compute(abuf, cbuf, sem, ch):
            sl = pl.ds(ch * chunk, chunk)
            pltpu.make_async_copy(table_hbm.at[sidx_all.at[sl]], abuf, sem).wait()
            pltpu.make_async_copy(c_hbm.at[didx_all.at[sl]], cbuf, sem).wait()

            def _group(g16, carry2):
                accs = [jnp.zeros((16,), jnp.float32) for _ in range(16)]
                hi_mask = jnp.full((16,), -65536, jnp.int32)
                for j2 in range(LANES // 2):
                    us0 = us[2 * j2]
                    us1 = us[2 * j2 + 1]
                    sl = pl.ds(16 * j2, 16)
                    for k in range(16):
                        g = g16 * 16 + k
                        wa = abuf[g, sl]
                        wc = cbuf[g, sl]
                        a0 = lax.bitcast_convert_type(wa << 16, jnp.float32)
                        c0 = lax.bitcast_convert_type(wc << 16, jnp.float32)
                        a1 = lax.bitcast_convert_type(wa & hi_mask, jnp.float32)
                        c1 = lax.bitcast_convert_type(wc & hi_mask, jnp.float32)
                        r0 = jnp.maximum(a0 + c0, 0.0)
                        r1 = jnp.maximum(a1 + c1, 0.0)
                        accs[k] = accs[k] + r0 * us0 + r1 * us1
                res = jnp.zeros((16,), jnp.float32)
                for k in range(16):
                    a = accs[k]
                    for sh in (8, 4, 2, 1):
                        a = a + _lane_shuffle(a, lane_iota ^ sh)
                    res = jnp.where(lane_iota == k, a + usb_all, res)
                outbuf[pl.ds(ch * chunk + g16 * 16, 16)] = res
                return carry2

            lax.fori_loop(0, chunk // 16, _group, 0)

        _start(a_a, c_a, sem_a, 0)

        def _pair(k, carry):
            _start(a_b, c_b, sem_b, 2 * k + 1)
            _compute(a_a, c_a, sem_a, 2 * k)

            @pl.when(k < n_chunks // 2 - 1)
            def _():
                _start(a_a, c_a, sem_a, 2 * k + 2)

            _compute(a_b, c_b, sem_b, 2 * k + 1)
            return carry

        lax.fori_loop(0, n_chunks // 2, _pair, 0)
        pltpu.sync_copy(outbuf, out_hbm.at[pl.ds(base, EP // 32)])

    return body


def kernel(ego_node, features, emb, edge_index, fc_w, fc_b, lin_w, lin_b, us_w, us_b):
    del ego_node
    features = features.astype(jnp.float32)
    emb = emb.astype(jnp.float32)

    l1 = lin_w[:H]
    l2 = lin_w[H:2 * H]
    l3 = lin_w[2 * H:]
    linb_row = lin_b.reshape(1, H)
    fcb2_row = (2.0 * fc_b).reshape(1, H)

    rows_blk = 1000
    grid = (N // rows_blk,)
    full = lambda shape: pl.BlockSpec(shape, lambda i: (0,) * len(shape))
    rblk = lambda width: pl.BlockSpec((rows_blk, width), lambda i: (i, 0))

    a_tab, c_tab, fcf = pl.pallas_call(
        _dense_pre_body,
        grid=grid,
        in_specs=[rblk(H), rblk(H), full((H, H)), full((H, H)), full((H, H)),
                  full((1, H)), full((1, H))],
        out_specs=[rblk(H), rblk(H), rblk(H)],
        out_shape=[jax.ShapeDtypeStruct((N, H), jnp.bfloat16),
                   jax.ShapeDtypeStruct((N, H), jnp.bfloat16),
                   jax.ShapeDtypeStruct((N, H), jnp.float32)],
    )(emb, features, l1, l3, fc_w, linb_row, fcb2_row)

    s0 = edge_index[0, :E2]
    d0 = edge_index[1, :E2]
    s1 = edge_index[0, E2:]
    d1 = edge_index[1, E2:]
    zpad = jnp.zeros((EP - E2,), jnp.int32)
    s0p = jnp.concatenate([s0, zpad])
    d0p = jnp.concatenate([d0, zpad])
    s1p = jnp.concatenate([s1, zpad])
    d1p = jnp.concatenate([d1, zpad])
    d0seg = jnp.concatenate([d0, jnp.full((EP - E2,), N, jnp.int32)])

    emb2 = jnp.concatenate([emb[:, :128], emb[:, 128:]], axis=0)

    mesh = plsc.VectorSubcoreMesh(core_axis_name="c", subcore_axis_name="s")

    n_seg_chunks = EP // 16 // _SEG_CHUNK
    s0p3 = s0p.reshape(16, n_seg_chunks, _SEG_CHUNK)
    d0seg3 = d0seg.reshape(16, n_seg_chunks, _SEG_CHUNK)

    seg = pl.kernel(
        _segment_body,
        out_type=[
            jax.ShapeDtypeStruct((2, NP, 128), jnp.float32),
            jax.ShapeDtypeStruct((2, NP, 128), jnp.float32),
        ],
        mesh=mesh,
        scratch_types=[
            pltpu.VMEM_SHARED((NP, 128), jnp.float32),
            pltpu.VMEM((n_seg_chunks, _SEG_CHUNK), jnp.int32),
            pltpu.VMEM((n_seg_chunks, _SEG_CHUNK), jnp.int32),
            pltpu.VMEM((_SEG_CHUNK, 128), jnp.float32),
            pltpu.VMEM((_SEG_CHUNK, 128), jnp.float32),
            pltpu.SemaphoreType.DMA,
            pltpu.SemaphoreType.DMA,
        ],
    )
    sr, deg2 = seg(emb2, s0p3, d0seg3)
    sr_lo, sr_hi = sr[0, :N], sr[1, :N]

    usw_blocks = us_w.reshape(H // 32, 16, 2)
    usw_flat = jnp.concatenate(
        [usw_blocks[:, :, 0], usw_blocks[:, :, 1]], axis=1).reshape(H)
    usb_pad = jnp.pad(us_b.astype(jnp.float32), (0, 15))

    pack2 = lambda t: lax.bitcast_convert_type(
        t.reshape(N, H // 2, 2), jnp.int32)
    a_pack = pack2(a_tab)
    c_pack = pack2(c_tab)

    score_scratch = [
        pltpu.VMEM((H,), jnp.float32),
        pltpu.VMEM((16,), jnp.float32),
        pltpu.VMEM((EP // 32,), jnp.int32),
        pltpu.VMEM((EP // 32,), jnp.int32),
        pltpu.VMEM((64, H // 2), jnp.int32),
        pltpu.VMEM((64, H // 2), jnp.int32),
        pltpu.VMEM((64, H // 2), jnp.int32),
        pltpu.VMEM((64, H // 2), jnp.int32),
        pltpu.VMEM((EP // 32,), jnp.float32),
        pltpu.SemaphoreType.DMA,
        pltpu.SemaphoreType.DMA,
    ]
    score0_fn = pl.kernel(
        _make_score_body(),
        out_type=[jax.ShapeDtypeStruct((EP,), jnp.float32)],
        mesh=mesh,
        scratch_types=score_scratch,
    )
    sc0, = score0_fn(a_pack, c_pack, s0p, d0p, usw_flat, usb_pad)

    p_tab, = pl.pallas_call(
        _dense_post_body,
        grid=grid,
        in_specs=[rblk(128), rblk(128), rblk(128), rblk(128), rblk(H), rblk(H),
                  full((128, H)), full((128, H)), full((H, H)), full((1, H))],
        out_specs=[rblk(H)],
        out_shape=[jax.ShapeDtypeStruct((N, H), jnp.bfloat16)],
    )(sr_lo, sr_hi, deg2[0, :N], deg2[1, :N], emb, fcf,
      l1[:128], l1[128:], l2, linb_row)

    score1_fn = pl.kernel(
        _make_score_body(),
        out_type=[jax.ShapeDtypeStruct((EP,), jnp.float32)],
        mesh=mesh,
        scratch_types=score_scratch,
    )
    sc1, = score1_fn(pack2(p_tab), c_pack, s1p, d1p, usw_flat, usb_pad)

    return (sc0[:E2].reshape(E2, 1), sc1[:E2].reshape(E2, 1))


# speedup vs baseline: 1.2037x; 1.2037x over previous
import jax
import jax.numpy as jnp
from jax import lax
from jax.experimental import pallas as pl
from jax.experimental.pallas import tpu as pltpu
from jax.experimental.pallas import tpu_sc as plsc

N = 10000
H = 256
E2 = 80000
EP = 81920
NP = 10240
LANES = 16
_SEG_CHUNK = 128
_SC_CHUNK = 64

_GATHER_DNUMS = lax.GatherDimensionNumbers(
    offset_dims=(), collapsed_slice_dims=(0,), start_index_map=(0,))


def _lane_shuffle(v, idx):
    return lax.gather(v, idx[:, None], _GATHER_DNUMS, (1,),
                      mode=lax.GatherScatterMode.PROMISE_IN_BOUNDS)


def _bf16_bits(x32):
    u = lax.bitcast_convert_type(x32, jnp.int32)
    return lax.shift_right_logical(u + 0x7FFF + ((u >> 16) & 1), 16)


def _pack_pair(xe, xo):
    return _bf16_bits(xe) | (_bf16_bits(xo) << 16)


def _dense_pre_body(emb_ref, f_ref, l1e_ref, l1o_ref, l3e_ref, l3o_ref,
                    fcw_ref, linbe_ref, linbo_ref, fcb2_ref,
                    a_ref, c_ref, fcf_ref):
    dot = lambda x, w: jnp.dot(x, w, preferred_element_type=jnp.float32)
    a_ref[...] = _pack_pair(dot(emb_ref[...], l1e_ref[...]) + linbe_ref[...],
                            dot(emb_ref[...], l1o_ref[...]) + linbo_ref[...])
    c_ref[...] = _pack_pair(dot(f_ref[...], l3e_ref[...]),
                            dot(f_ref[...], l3o_ref[...]))
    fcf_ref[...] = dot(f_ref[...], fcw_ref[...]) + fcb2_ref[...]


def _dense_post_body(srlo_ref, srhi_ref, dega_ref, degb_ref, emb_ref, fcf_ref,
                     l1ae_ref, l1ao_ref, l1be_ref, l1bo_ref,
                     l2e_ref, l2o_ref, linbe_ref, linbo_ref, p_ref):
    deg = dega_ref[:, 0:1] + degb_ref[:, 0:1]
    recv = deg > 0.0
    denom = jnp.maximum(deg, 1.0)
    rlo = jnp.where(recv, srlo_ref[...] / denom, emb_ref[:, :128])
    rhi = jnp.where(recv, srhi_ref[...] / denom, emb_ref[:, 128:])
    m1 = jnp.where(recv, jnp.maximum(fcf_ref[...], 0.0), 0.0)
    dot = lambda x, w: jnp.dot(x, w, preferred_element_type=jnp.float32)
    pe = (dot(rlo, l1ae_ref[...]) + dot(rhi, l1be_ref[...])
          + dot(m1, l2e_ref[...]) + linbe_ref[...])
    po = (dot(rlo, l1ao_ref[...]) + dot(rhi, l1bo_ref[...])
          + dot(m1, l2o_ref[...]) + linbo_ref[...])
    p_ref[...] = _pack_pair(pe, po)


def _segment_body(emb2_hbm, s3_hbm, d3_hbm,
                  sr_hbm, deg_hbm,
                  acc, sidx_all, didx_all, rows_a, rows_b, sem_a, sem_b):
    c = lax.axis_index("c")
    s = lax.axis_index("s")
    n_chunks = EP // 16 // _SEG_CHUNK

    def _init(r, carry):
        for j in range(8):
            rows_a[r, pl.ds(16 * j, 16)] = jnp.zeros((16,), jnp.float32)
        return carry

    lax.fori_loop(0, _SEG_CHUNK, _init, 0)

    for i in range(5):
        pltpu.sync_copy(rows_a, acc.at[pl.ds(s * 640 + i * 128, 128)])

    pltpu.sync_copy(s3_hbm.at[s], sidx_all)
    pltpu.sync_copy(d3_hbm.at[s], didx_all)
    row_off = c * N

    def _off(j, carry):
        for i in range(_SEG_CHUNK // 16):
            sl = pl.ds(16 * i, 16)
            sidx_all[j, sl] = sidx_all[j, sl] + row_off
        return carry

    lax.fori_loop(0, n_chunks, _off, 0)
    plsc.subcore_barrier()

    def _start(buf, sem, ch):
        pltpu.async_copy(emb2_hbm.at[sidx_all.at[ch]], buf, sem)

    def _finish(buf, sem, ch):
        pltpu.make_async_copy(emb2_hbm.at[sidx_all.at[ch]], buf, sem).wait()
        pltpu.sync_copy(buf, acc.at[didx_all.at[ch]], add=True)

    _start(rows_a, sem_a, 0)

    def _pair(k, carry):
        _start(rows_b, sem_b, 2 * k + 1)
        _finish(rows_a, sem_a, 2 * k)

        @pl.when(k < n_chunks // 2 - 1)
        def _():
            _start(rows_a, sem_a, 2 * k + 2)

        _finish(rows_b, sem_b, 2 * k + 1)
        return carry

    lax.fori_loop(0, n_chunks // 2, _pair, 0)
    plsc.subcore_barrier()

    rb = s * (NP // 16)
    pltpu.sync_copy(acc.at[pl.ds(rb, NP // 16)],
                    sr_hbm.at[c, pl.ds(rb, NP // 16)])

    def _fill(r, carry):
        for j in range(8):
            rows_b[r, pl.ds(16 * j, 16)] = jnp.zeros((16,), jnp.float32)
            rows_a[r, pl.ds(16 * j, 16)] = jnp.ones((16,), jnp.float32)
        return carry

    lax.fori_loop(0, _SEG_CHUNK, _fill, 0)
    for i in range(5):
        pltpu.sync_copy(rows_b, acc.at[pl.ds(s * 640 + i * 128, 128)])
    plsc.subcore_barrier()

    def _count(i, carry):
        ch = c * (n_chunks // 2) + i
        pltpu.sync_copy(rows_a, acc.at[didx_all.at[ch]], add=True)
        return carry

    lax.fori_loop(0, n_chunks // 2, _count, 0)
    plsc.subcore_barrier()
    pltpu.sync_copy(acc.at[pl.ds(rb, NP // 16)],
                    deg_hbm.at[c, pl.ds(rb, NP // 16)])


def _make_score_body():
    chunk = 64
    n_chunks = (EP // 32) // chunk

    def body(*refs):
        (table_hbm, c_hbm, s_hbm, d_hbm, usw_hbm, usb_hbm,
         out_hbm,
         usv, usbv, sidx_all, didx_all,
         a_a, c_a, a_b, c_b, outbuf, sem_a, sem_b) = refs

        cc = lax.axis_index("c")
        ss = lax.axis_index("s")
        w = ss * 2 + cc
        base = w * (EP // 32)

        pltpu.sync_copy(usw_hbm, usv)
        pltpu.sync_copy(usb_hbm, usbv)
        pltpu.sync_copy(s_hbm.at[pl.ds(base, EP // 32)], sidx_all)
        pltpu.sync_copy(d_hbm.at[pl.ds(base, EP // 32)], didx_all)
        us = [usv[pl.ds(16 * j, 16)] for j in range(LANES)]
        lane_iota = lax.broadcasted_iota(jnp.int32, (16,), 0)
        usb_all = _lane_shuffle(usbv[...], lane_iota * 0)

        def _start(abuf, cbuf, sem, ch):
            sl = pl.ds(ch * chunk, chunk)
            pltpu.async_copy(table_hbm.at[sidx_all.at[sl]], abuf, sem)
            pltpu.async_copy(c_hbm.at[didx_all.at[sl]], cbuf, sem)

        def _compute(abuf, cbuf, sem, ch):
            sl = pl.ds(ch * chunk, chunk)
            pltpu.make_async_copy(table_hbm.at[sidx_all.at[sl]], abuf, sem).wait()
            pltpu.make_async_copy(c_hbm.at[didx_all.at[sl]], cbuf, sem).wait()

            def _group(g16, carry2):
                accs = [jnp.zeros((16,), jnp.float32) for _ in range(16)]
                hi_mask = jnp.full((16,), -65536, jnp.int32)
                for j2 in range(LANES // 2):
                    us0 = us[2 * j2]
                    us1 = us[2 * j2 + 1]
                    sl = pl.ds(16 * j2, 16)
                    for k in range(16):
                        g = g16 * 16 + k
                        wa = abuf[g, sl]
                        wc = cbuf[g, sl]
                        a0 = lax.bitcast_convert_type(wa << 16, jnp.float32)
                        c0 = lax.bitcast_convert_type(wc << 16, jnp.float32)
                        a1 = lax.bitcast_convert_type(wa & hi_mask, jnp.float32)
                        c1 = lax.bitcast_convert_type(wc & hi_mask, jnp.float32)
                        r0 = jnp.maximum(a0 + c0, 0.0)
                        r1 = jnp.maximum(a1 + c1, 0.0)
                        accs[k] = accs[k] + r0 * us0 + r1 * us1
                res = jnp.zeros((16,), jnp.float32)
                for k in range(16):
                    a = accs[k]
                    for sh in (8, 4, 2, 1):
                        a = a + _lane_shuffle(a, lane_iota ^ sh)
                    res = jnp.where(lane_iota == k, a + usb_all, res)
                outbuf[pl.ds(ch * chunk + g16 * 16, 16)] = res
                return carry2

            lax.fori_loop(0, chunk // 16, _group, 0)

        _start(a_a, c_a, sem_a, 0)

        def _pair(k, carry):
            _start(a_b, c_b, sem_b, 2 * k + 1)
            _compute(a_a, c_a, sem_a, 2 * k)

            @pl.when(k < n_chunks // 2 - 1)
            def _():
                _start(a_a, c_a, sem_a, 2 * k + 2)

            _compute(a_b, c_b, sem_b, 2 * k + 1)
            return carry

        lax.fori_loop(0, n_chunks // 2, _pair, 0)
        pltpu.sync_copy(outbuf, out_hbm.at[pl.ds(base, EP // 32)])

    return body


def kernel(ego_node, features, emb, edge_index, fc_w, fc_b, lin_w, lin_b, us_w, us_b):
    del ego_node
    features = features.astype(jnp.float32)
    emb = emb.astype(jnp.float32)

    l1 = lin_w[:H]
    l2 = lin_w[H:2 * H]
    l3 = lin_w[2 * H:]
    fcb2_row = (2.0 * fc_b).reshape(1, H)
    ev = lambda w: w[:, 0::2]
    od = lambda w: w[:, 1::2]
    linbe_row = lin_b[0::2].reshape(1, H // 2)
    linbo_row = lin_b[1::2].reshape(1, H // 2)

    rows_blk = 1000
    grid = (N // rows_blk,)
    full = lambda shape: pl.BlockSpec(shape, lambda i: (0,) * len(shape))
    rblk = lambda width: pl.BlockSpec((rows_blk, width), lambda i: (i, 0))

    a_tab, c_tab, fcf = pl.pallas_call(
        _dense_pre_body,
        grid=grid,
        in_specs=[rblk(H), rblk(H),
                  full((H, H // 2)), full((H, H // 2)),
                  full((H, H // 2)), full((H, H // 2)),
                  full((H, H)), full((1, H // 2)), full((1, H // 2)),
                  full((1, H))],
        out_specs=[rblk(H // 2), rblk(H // 2), rblk(H)],
        out_shape=[jax.ShapeDtypeStruct((N, H // 2), jnp.int32),
                   jax.ShapeDtypeStruct((N, H // 2), jnp.int32),
                   jax.ShapeDtypeStruct((N, H), jnp.float32)],
    )(emb, features, ev(l1), od(l1), ev(l3), od(l3), fc_w,
      linbe_row, linbo_row, fcb2_row)

    s0 = edge_index[0, :E2]
    d0 = edge_index[1, :E2]
    s1 = edge_index[0, E2:]
    d1 = edge_index[1, E2:]
    zpad = jnp.zeros((EP - E2,), jnp.int32)
    s0p = jnp.concatenate([s0, zpad])
    d0p = jnp.concatenate([d0, zpad])
    s1p = jnp.concatenate([s1, zpad])
    d1p = jnp.concatenate([d1, zpad])
    d0seg = jnp.concatenate([d0, jnp.full((EP - E2,), N, jnp.int32)])

    emb2 = jnp.concatenate([emb[:, :128], emb[:, 128:]], axis=0)

    mesh = plsc.VectorSubcoreMesh(core_axis_name="c", subcore_axis_name="s")

    n_seg_chunks = EP // 16 // _SEG_CHUNK
    s0p3 = s0p.reshape(16, n_seg_chunks, _SEG_CHUNK)
    d0seg3 = d0seg.reshape(16, n_seg_chunks, _SEG_CHUNK)

    seg = pl.kernel(
        _segment_body,
        out_type=[
            jax.ShapeDtypeStruct((2, NP, 128), jnp.float32),
            jax.ShapeDtypeStruct((2, NP, 128), jnp.float32),
        ],
        mesh=mesh,
        scratch_types=[
            pltpu.VMEM_SHARED((NP, 128), jnp.float32),
            pltpu.VMEM((n_seg_chunks, _SEG_CHUNK), jnp.int32),
            pltpu.VMEM((n_seg_chunks, _SEG_CHUNK), jnp.int32),
            pltpu.VMEM((_SEG_CHUNK, 128), jnp.float32),
            pltpu.VMEM((_SEG_CHUNK, 128), jnp.float32),
            pltpu.SemaphoreType.DMA,
            pltpu.SemaphoreType.DMA,
        ],
    )
    sr, deg2 = seg(emb2, s0p3, d0seg3)
    sr_lo, sr_hi = sr[0, :N], sr[1, :N]

    usw_blocks = us_w.reshape(H // 32, 16, 2)
    usw_flat = jnp.concatenate(
        [usw_blocks[:, :, 0], usw_blocks[:, :, 1]], axis=1).reshape(H)
    usb_pad = jnp.pad(us_b.astype(jnp.float32), (0, 15))

    score_scratch = [
        pltpu.VMEM((H,), jnp.float32),
        pltpu.VMEM((16,), jnp.float32),
        pltpu.VMEM((EP // 32,), jnp.int32),
        pltpu.VMEM((EP // 32,), jnp.int32),
        pltpu.VMEM((64, H // 2), jnp.int32),
        pltpu.VMEM((64, H // 2), jnp.int32),
        pltpu.VMEM((64, H // 2), jnp.int32),
        pltpu.VMEM((64, H // 2), jnp.int32),
        pltpu.VMEM((EP // 32,), jnp.float32),
        pltpu.SemaphoreType.DMA,
        pltpu.SemaphoreType.DMA,
    ]
    score0_fn = pl.kernel(
        _make_score_body(),
        out_type=[jax.ShapeDtypeStruct((EP,), jnp.float32)],
        mesh=mesh,
        scratch_types=score_scratch,
    )
    sc0, = score0_fn(a_tab, c_tab, s0p, d0p, usw_flat, usb_pad)

    p_tab, = pl.pallas_call(
        _dense_post_body,
        grid=grid,
        in_specs=[rblk(128), rblk(128), rblk(128), rblk(128), rblk(H), rblk(H),
                  full((128, H // 2)), full((128, H // 2)),
                  full((128, H // 2)), full((128, H // 2)),
                  full((H, H // 2)), full((H, H // 2)),
                  full((1, H // 2)), full((1, H // 2))],
        out_specs=[rblk(H // 2)],
        out_shape=[jax.ShapeDtypeStruct((N, H // 2), jnp.int32)],
    )(sr_lo, sr_hi, deg2[0, :N], deg2[1, :N], emb, fcf,
      ev(l1[:128]), od(l1[:128]), ev(l1[128:]), od(l1[128:]),
      ev(l2), od(l2), linbe_row, linbo_row)

    score1_fn = pl.kernel(
        _make_score_body(),
        out_type=[jax.ShapeDtypeStruct((EP,), jnp.float32)],
        mesh=mesh,
        scratch_types=score_scratch,
    )
    sc1, = score1_fn(p_tab, c_tab, s1p, d1p, usw_flat, usb_pad)

    return (sc0[:E2].reshape(E2, 1), sc1[:E2].reshape(E2, 1))


# speedup vs baseline: 1.2041x; 1.0004x over previous
import jax
import jax.numpy as jnp
from jax import lax
from jax.experimental import pallas as pl
from jax.experimental.pallas import tpu as pltpu
from jax.experimental.pallas import tpu_sc as plsc

N = 10000
H = 256
E2 = 80000
EP = 81920
NP = 10240
LANES = 16
_SEG_CHUNK = 128
_SC_CHUNK = 64

_GATHER_DNUMS = lax.GatherDimensionNumbers(
    offset_dims=(), collapsed_slice_dims=(0,), start_index_map=(0,))


def _lane_shuffle(v, idx):
    return lax.gather(v, idx[:, None], _GATHER_DNUMS, (1,),
                      mode=lax.GatherScatterMode.PROMISE_IN_BOUNDS)


def _bf16_bits(x32):
    u = lax.bitcast_convert_type(x32, jnp.int32)
    return lax.shift_right_logical(u + 0x7FFF + ((u >> 16) & 1), 16)


def _pack_pair(xe, xo):
    return _bf16_bits(xe) | (_bf16_bits(xo) << 16)


def _dense_pre_body(emb_ref, f_ref, l1e_ref, l1o_ref, l3e_ref, l3o_ref,
                    fcw_ref, linbe_ref, linbo_ref, fcb2_ref,
                    a_ref, c_ref, fcf_ref):
    dot = lambda x, w: jnp.dot(x, w, preferred_element_type=jnp.float32)
    a_ref[...] = _pack_pair(dot(emb_ref[...], l1e_ref[...]) + linbe_ref[...],
                            dot(emb_ref[...], l1o_ref[...]) + linbo_ref[...])
    c_ref[...] = _pack_pair(dot(f_ref[...], l3e_ref[...]),
                            dot(f_ref[...], l3o_ref[...]))
    fcf_ref[...] = dot(f_ref[...], fcw_ref[...]) + fcb2_ref[...]


def _dense_post_body(srlo_ref, srhi_ref, dega_ref, degb_ref, emb_ref, fcf_ref,
                     l1ae_ref, l1ao_ref, l1be_ref, l1bo_ref,
                     l2e_ref, l2o_ref, linbe_ref, linbo_ref, p_ref):
    deg = dega_ref[:, 0:1] + degb_ref[:, 0:1]
    recv = deg > 0.0
    denom = jnp.maximum(deg, 1.0)
    rlo = jnp.where(recv, srlo_ref[...] / denom, emb_ref[:, :128])
    rhi = jnp.where(recv, srhi_ref[...] / denom, emb_ref[:, 128:])
    m1 = jnp.where(recv, jnp.maximum(fcf_ref[...], 0.0), 0.0)
    dot = lambda x, w: jnp.dot(x, w, preferred_element_type=jnp.float32)
    pe = (dot(rlo, l1ae_ref[...]) + dot(rhi, l1be_ref[...])
          + dot(m1, l2e_ref[...]) + linbe_ref[...])
    po = (dot(rlo, l1ao_ref[...]) + dot(rhi, l1bo_ref[...])
          + dot(m1, l2o_ref[...]) + linbo_ref[...])
    p_ref[...] = _pack_pair(pe, po)


def _segment_body(emb2_hbm, s3_hbm, d3_hbm,
                  sr_hbm, deg_hbm,
                  acc, sidx_all, didx_all, rows_a, rows_b, sem_a, sem_b):
    c = lax.axis_index("c")
    s = lax.axis_index("s")
    n_chunks = EP // 16 // _SEG_CHUNK

    def _init(r, carry):
        for j in range(8):
            rows_a[r, pl.ds(16 * j, 16)] = jnp.zeros((16,), jnp.float32)
        return carry

    lax.fori_loop(0, _SEG_CHUNK, _init, 0)

    for i in range(5):
        pltpu.sync_copy(rows_a, acc.at[pl.ds(s * 640 + i * 128, 128)])

    pltpu.sync_copy(s3_hbm.at[s], sidx_all)
    pltpu.sync_copy(d3_hbm.at[s], didx_all)
    row_off = c * N

    def _off(j, carry):
        for i in range(_SEG_CHUNK // 16):
            sl = pl.ds(16 * i, 16)
            sidx_all[j, sl] = sidx_all[j, sl] + row_off
        return carry

    lax.fori_loop(0, n_chunks, _off, 0)
    plsc.subcore_barrier()

    def _start(buf, sem, ch):
        pltpu.async_copy(emb2_hbm.at[sidx_all.at[ch]], buf, sem)

    def _finish(buf, sem, ch):
        pltpu.make_async_copy(emb2_hbm.at[sidx_all.at[ch]], buf, sem).wait()
        pltpu.sync_copy(buf, acc.at[didx_all.at[ch]], add=True)

    _start(rows_a, sem_a, 0)

    def _pair(k, carry):
        _start(rows_b, sem_b, 2 * k + 1)
        _finish(rows_a, sem_a, 2 * k)

        @pl.when(k < n_chunks // 2 - 1)
        def _():
            _start(rows_a, sem_a, 2 * k + 2)

        _finish(rows_b, sem_b, 2 * k + 1)
        return carry

    lax.fori_loop(0, n_chunks // 2, _pair, 0)
    plsc.subcore_barrier()

    rb = s * (NP // 16)
    pltpu.sync_copy(acc.at[pl.ds(rb, NP // 16)],
                    sr_hbm.at[c, pl.ds(rb, NP // 16)])

    def _fill(r, carry):
        for j in range(8):
            rows_b[r, pl.ds(16 * j, 16)] = jnp.zeros((16,), jnp.float32)
            rows_a[r, pl.ds(16 * j, 16)] = jnp.ones((16,), jnp.float32)
        return carry

    lax.fori_loop(0, _SEG_CHUNK, _fill, 0)
    for i in range(5):
        pltpu.sync_copy(rows_b, acc.at[pl.ds(s * 640 + i * 128, 128)])
    plsc.subcore_barrier()

    def _count(i, carry):
        ch = c * (n_chunks // 2) + i
        pltpu.async_copy(rows_a, acc.at[didx_all.at[ch]], sem_a, add=True)
        return carry

    lax.fori_loop(0, n_chunks // 2, _count, 0)

    def _drain(i, carry):
        ch = c * (n_chunks // 2) + i
        pltpu.make_async_copy(rows_a, acc.at[didx_all.at[ch]], sem_a).wait()
        return carry

    lax.fori_loop(0, n_chunks // 2, _drain, 0)
    plsc.subcore_barrier()
    pltpu.sync_copy(acc.at[pl.ds(rb, NP // 16)],
                    deg_hbm.at[c, pl.ds(rb, NP // 16)])


def _make_score_body():
    chunk = 64
    n_chunks = (EP // 32) // chunk

    def body(*refs):
        (table_hbm, c_hbm, s_hbm, d_hbm, usw_hbm, usb_hbm,
         out_hbm,
         usv, usbv, sidx_all, didx_all,
         a_a, c_a, a_b, c_b, outbuf, sem_a, sem_b) = refs

        cc = lax.axis_index("c")
        ss = lax.axis_index("s")
        w = ss * 2 + cc
        base = w * (EP // 32)

        pltpu.sync_copy(usw_hbm, usv)
        pltpu.sync_copy(usb_hbm, usbv)
        pltpu.sync_copy(s_hbm.at[pl.ds(base, EP // 32)], sidx_all)
        pltpu.sync_copy(d_hbm.at[pl.ds(base, EP // 32)], didx_all)
        us = [usv[pl.ds(16 * j, 16)] for j in range(LANES)]
        lane_iota = lax.broadcasted_iota(jnp.int32, (16,), 0)
        usb_all = _lane_shuffle(usbv[...], lane_iota * 0)

        def _start(abuf, cbuf, sem, ch):
            sl = pl.ds(ch * chunk, chunk)
            pltpu.async_copy(table_hbm.at[sidx_all.at[sl]], abuf, sem)
            pltpu.async_copy(c_hbm.at[didx_all.at[sl]], cbuf, sem)

        def _compute(abuf, cbuf, sem, ch):
            sl = pl.ds(ch * chunk, chunk)
            pltpu.make_async_copy(table_hbm.at[sidx_all.at[sl]], abuf, sem).wait()
            pltpu.make_async_copy(c_hbm.at[didx_all.at[sl]], cbuf, sem).wait()

            def _group(g16, carry2):
                accs = [jnp.zeros((16,), jnp.float32) for _ in range(16)]
                hi_mask = jnp.full((16,), -65536, jnp.int32)
                for j2 in range(LANES // 2):
                    us0 = us[2 * j2]
                    us1 = us[2 * j2 + 1]
                    sl = pl.ds(16 * j2, 16)
                    for k in range(16):
                        g = g16 * 16 + k
                        wa = abuf[g, sl]
                        wc = cbuf[g, sl]
                        a0 = lax.bitcast_convert_type(wa << 16, jnp.float32)
                        c0 = lax.bitcast_convert_type(wc << 16, jnp.float32)
                        a1 = lax.bitcast_convert_type(wa & hi_mask, jnp.float32)
                        c1 = lax.bitcast_convert_type(wc & hi_mask, jnp.float32)
                        r0 = jnp.maximum(a0 + c0, 0.0)
                        r1 = jnp.maximum(a1 + c1, 0.0)
                        accs[k] = accs[k] + r0 * us0 + r1 * us1
                res = jnp.zeros((16,), jnp.float32)
                for k in range(16):
                    a = accs[k]
                    for sh in (8, 4, 2, 1):
                        a = a + _lane_shuffle(a, lane_iota ^ sh)
                    res = jnp.where(lane_iota == k, a + usb_all, res)
                outbuf[pl.ds(ch * chunk + g16 * 16, 16)] = res
                return carry2

            lax.fori_loop(0, chunk // 16, _group, 0)

        _start(a_a, c_a, sem_a, 0)

        def _pair(k, carry):
            _start(a_b, c_b, sem_b, 2 * k + 1)
            _compute(a_a, c_a, sem_a, 2 * k)

            @pl.when(k < n_chunks // 2 - 1)
            def _():
                _start(a_a, c_a, sem_a, 2 * k + 2)

            _compute(a_b, c_b, sem_b, 2 * k + 1)
            return carry

        lax.fori_loop(0, n_chunks // 2, _pair, 0)
        pltpu.sync_copy(outbuf, out_hbm.at[pl.ds(base, EP // 32)])

    return body


def kernel(ego_node, features, emb, edge_index, fc_w, fc_b, lin_w, lin_b, us_w, us_b):
    del ego_node
    features = features.astype(jnp.float32)
    emb = emb.astype(jnp.float32)

    l1 = lin_w[:H]
    l2 = lin_w[H:2 * H]
    l3 = lin_w[2 * H:]
    fcb2_row = (2.0 * fc_b).reshape(1, H)
    ev = lambda w: w[:, 0::2]
    od = lambda w: w[:, 1::2]
    linbe_row = lin_b[0::2].reshape(1, H // 2)
    linbo_row = lin_b[1::2].reshape(1, H // 2)

    rows_blk = 1000
    grid = (N // rows_blk,)
    full = lambda shape: pl.BlockSpec(shape, lambda i: (0,) * len(shape))
    rblk = lambda width: pl.BlockSpec((rows_blk, width), lambda i: (i, 0))

    a_tab, c_tab, fcf = pl.pallas_call(
        _dense_pre_body,
        grid=grid,
        in_specs=[rblk(H), rblk(H),
                  full((H, H // 2)), full((H, H // 2)),
                  full((H, H // 2)), full((H, H // 2)),
                  full((H, H)), full((1, H // 2)), full((1, H // 2)),
                  full((1, H))],
        out_specs=[rblk(H // 2), rblk(H // 2), rblk(H)],
        out_shape=[jax.ShapeDtypeStruct((N, H // 2), jnp.int32),
                   jax.ShapeDtypeStruct((N, H // 2), jnp.int32),
                   jax.ShapeDtypeStruct((N, H), jnp.float32)],
    )(emb, features, ev(l1), od(l1), ev(l3), od(l3), fc_w,
      linbe_row, linbo_row, fcb2_row)

    s0 = edge_index[0, :E2]
    d0 = edge_index[1, :E2]
    s1 = edge_index[0, E2:]
    d1 = edge_index[1, E2:]
    zpad = jnp.zeros((EP - E2,), jnp.int32)
    s0p = jnp.concatenate([s0, zpad])
    d0p = jnp.concatenate([d0, zpad])
    s1p = jnp.concatenate([s1, zpad])
    d1p = jnp.concatenate([d1, zpad])
    d0seg = jnp.concatenate([d0, jnp.full((EP - E2,), N, jnp.int32)])

    emb2 = jnp.concatenate([emb[:, :128], emb[:, 128:]], axis=0)

    mesh = plsc.VectorSubcoreMesh(core_axis_name="c", subcore_axis_name="s")

    n_seg_chunks = EP // 16 // _SEG_CHUNK
    s0p3 = s0p.reshape(16, n_seg_chunks, _SEG_CHUNK)
    d0seg3 = d0seg.reshape(16, n_seg_chunks, _SEG_CHUNK)

    seg = pl.kernel(
        _segment_body,
        out_type=[
            jax.ShapeDtypeStruct((2, NP, 128), jnp.float32),
            jax.ShapeDtypeStruct((2, NP, 128), jnp.float32),
        ],
        mesh=mesh,
        scratch_types=[
            pltpu.VMEM_SHARED((NP, 128), jnp.float32),
            pltpu.VMEM((n_seg_chunks, _SEG_CHUNK), jnp.int32),
            pltpu.VMEM((n_seg_chunks, _SEG_CHUNK), jnp.int32),
            pltpu.VMEM((_SEG_CHUNK, 128), jnp.float32),
            pltpu.VMEM((_SEG_CHUNK, 128), jnp.float32),
            pltpu.SemaphoreType.DMA,
            pltpu.SemaphoreType.DMA,
        ],
    )
    sr, deg2 = seg(emb2, s0p3, d0seg3)
    sr_lo, sr_hi = sr[0, :N], sr[1, :N]

    usw_blocks = us_w.reshape(H // 32, 16, 2)
    usw_flat = jnp.concatenate(
        [usw_blocks[:, :, 0], usw_blocks[:, :, 1]], axis=1).reshape(H)
    usb_pad = jnp.pad(us_b.astype(jnp.float32), (0, 15))

    score_scratch = [
        pltpu.VMEM((H,), jnp.float32),
        pltpu.VMEM((16,), jnp.float32),
        pltpu.VMEM((EP // 32,), jnp.int32),
        pltpu.VMEM((EP // 32,), jnp.int32),
        pltpu.VMEM((64, H // 2), jnp.int32),
        pltpu.VMEM((64, H // 2), jnp.int32),
        pltpu.VMEM((64, H // 2), jnp.int32),
        pltpu.VMEM((64, H // 2), jnp.int32),
        pltpu.VMEM((EP // 32,), jnp.float32),
        pltpu.SemaphoreType.DMA,
        pltpu.SemaphoreType.DMA,
    ]
    score0_fn = pl.kernel(
        _make_score_body(),
        out_type=[jax.ShapeDtypeStruct((EP,), jnp.float32)],
        mesh=mesh,
        scratch_types=score_scratch,
    )
    sc0, = score0_fn(a_tab, c_tab, s0p, d0p, usw_flat, usb_pad)

    p_tab, = pl.pallas_call(
        _dense_post_body,
        grid=grid,
        in_specs=[rblk(128), rblk(128), rblk(128), rblk(128), rblk(H), rblk(H),
                  full((128, H // 2)), full((128, H // 2)),
                  full((128, H // 2)), full((128, H // 2)),
                  full((H, H // 2)), full((H, H // 2)),
                  full((1, H // 2)), full((1, H // 2))],
        out_specs=[rblk(H // 2)],
        out_shape=[jax.ShapeDtypeStruct((N, H // 2), jnp.int32)],
    )(sr_lo, sr_hi, deg2[0, :N], deg2[1, :N], emb, fcf,
      ev(l1[:128]), od(l1[:128]), ev(l1[128:]), od(l1[128:]),
      ev(l2), od(l2), linbe_row, linbo_row)

    score1_fn = pl.kernel(
        _make_score_body(),
        out_type=[jax.ShapeDtypeStruct((EP,), jnp.float32)],
        mesh=mesh,
        scratch_types=score_scratch,
    )
    sc1, = score1_fn(p_tab, c_tab, s1p, d1p, usw_flat, usb_pad)

    return (sc0[:E2].reshape(E2, 1), sc1[:E2].reshape(E2, 1))
